# baseline (speedup 1.0000x reference)
import functools

import jax
import jax.numpy as jnp
from jax import lax
from jax.experimental import pallas as pl
from jax.experimental.pallas import tpu as pltpu
from jax.experimental.pallas import tpu_sc as plsc

B = 16
IMG = 224
PATCH = 16
CIN = 3
EMB = 384
HID = 1536
NEXP = 8
NCLS = 1000
NPATCH = (IMG // PATCH) ** 2
NTOK = NPATCH + 1
T = B * NTOK
TILE = 128
TP = 3328
NP_PAD = 3200
A_PAD = 58 * TILE
TRASH = A_PAD - 1
NW = 32
NEG = -3.0e38


def _patch_body(p_ref, w_ref, b_ref, o_ref):
    o_ref[...] = lax.dot_general(
        p_ref[...], w_ref[...], (((1,), (1,)), ((), ())),
        preferred_element_type=jnp.float32) + b_ref[0]


def _router_body(x_ref, w_ref, b_ref, o_ref):
    logits = lax.dot_general(
        x_ref[...], w_ref[...], (((1,), (1,)), ((), ())),
        preferred_element_type=jnp.float32) + b_ref[0]
    lane = lax.broadcasted_iota(jnp.int32, (TILE, 128), 1)
    logits = jnp.where(lane < NEXP, logits, NEG)
    m0 = jnp.max(logits, axis=1, keepdims=True)
    i0 = jnp.min(jnp.where(logits >= m0, lane, 128), axis=1, keepdims=True)
    oh0 = lane == i0
    l2 = jnp.where(oh0, NEG, logits)
    m1 = jnp.max(l2, axis=1, keepdims=True)
    i1 = jnp.min(jnp.where(l2 >= m1, lane, 128), axis=1, keepdims=True)
    mask = jnp.logical_or(oh0, lane == i1)
    o_ref[...] = mask[:, :NEXP].astype(jnp.int32)


def _ln(x, g, b, eps=1e-5):
    mu = jnp.mean(x, axis=1, keepdims=True)
    var = jnp.mean((x - mu) ** 2, axis=1, keepdims=True)
    return (x - mu) * lax.rsqrt(var + eps) * g + b


def _ffn_body(eot_ref, xs_ref, g1_ref, c1_ref, wv_ref, bv_ref, wo_ref, bo_ref,
              g2_ref, c2_ref, w1_ref, b1_ref, w2_ref, b2_ref, ys_ref):
    nt = (((1,), (1,)), ((), ()))
    bf = jnp.bfloat16
    x = xs_ref[...]
    xn = _ln(x, g1_ref[0], c1_ref[0])
    v = lax.dot_general(xn.astype(bf), wv_ref[0], nt,
                        preferred_element_type=jnp.float32) + bv_ref[0]
    attn = lax.dot_general(v.astype(bf), wo_ref[0], nt,
                           preferred_element_type=jnp.float32) + bo_ref[0]
    hmid = x + attn
    hn = _ln(hmid, g2_ref[0], c2_ref[0])
    h1 = lax.dot_general(hn.astype(bf), w1_ref[0], nt,
                         preferred_element_type=jnp.float32) + b1_ref[0]
    h1 = 0.5 * h1 * (1.0 + lax.erf(h1 * 0.7071067811865476))
    m = lax.dot_general(h1.astype(bf), w2_ref[0], nt,
                        preferred_element_type=jnp.float32) + b2_ref[0]
    ys_ref[...] = 0.5 * (hmid + m)


def _head_body(x_ref, g_ref, b_ref, w_ref, hb_ref, o_ref):
    xn = _ln(x_ref[...], g_ref[0], b_ref[0])
    o_ref[...] = lax.dot_general(
        xn, w_ref[...], (((1,), (1,)), ((), ())),
        preferred_element_type=jnp.float32) + hb_ref[0]


@functools.lru_cache(maxsize=None)
def _sc_dispatch_kernel():
    mesh = plsc.VectorSubcoreMesh(core_axis_name="c", subcore_axis_name="s")

    @functools.partial(
        pl.kernel,
        out_type=jax.ShapeDtypeStruct((A_PAD, EMB), jnp.float32),
        mesh=mesh,
        scratch_types=[
            pltpu.VMEM((TP // NW,), jnp.int32),
            pltpu.VMEM((TP // NW,), jnp.int32),
            pltpu.VMEM((TP // NW, EMB), jnp.float32),
            pltpu.SemaphoreType.DMA,
        ],
    )
    def disp(tok_hbm, d0_hbm, d1_hbm, out_hbm, i0_v, i1_v, rows_v, sem):
        n = TP // NW
        wid = lax.axis_index("s") * 2 + lax.axis_index("c")
        base = wid * n
        pltpu.sync_copy(d0_hbm.at[pl.ds(base, n)], i0_v)
        pltpu.sync_copy(d1_hbm.at[pl.ds(base, n)], i1_v)
        pltpu.sync_copy(tok_hbm.at[pl.ds(base, n)], rows_v)
        pltpu.async_copy(rows_v, out_hbm.at[i0_v], sem).wait()
        pltpu.async_copy(rows_v, out_hbm.at[i1_v], sem).wait()

    return disp


def _sc_dispatch(tok, d0, d1):
    return _sc_dispatch_kernel()(tok, d0, d1)


@functools.lru_cache(maxsize=None)
def _sc_combine_kernel():
    mesh = plsc.VectorSubcoreMesh(core_axis_name="c", subcore_axis_name="s")

    @functools.partial(
        pl.kernel,
        out_type=jax.ShapeDtypeStruct((TP, EMB), jnp.float32),
        mesh=mesh,
        scratch_types=[
            pltpu.VMEM((TP // NW,), jnp.int32),
            pltpu.VMEM((TP // NW,), jnp.int32),
            pltpu.VMEM((TP // NW, EMB), jnp.float32),
            pltpu.VMEM((TP // NW, EMB), jnp.float32),
            pltpu.SemaphoreType.DMA,
        ],
    )
    def comb(ys_hbm, d0_hbm, d1_hbm, out_hbm, i0_v, i1_v, r0_v, r1_v, sem):
        n = TP // NW
        wid = lax.axis_index("s") * 2 + lax.axis_index("c")
        base = wid * n
        pltpu.sync_copy(d0_hbm.at[pl.ds(base, n)], i0_v)
        pltpu.sync_copy(d1_hbm.at[pl.ds(base, n)], i1_v)
        pltpu.async_copy(ys_hbm.at[i0_v], r0_v, sem).wait()
        pltpu.async_copy(ys_hbm.at[i1_v], r1_v, sem).wait()

        def row(r, carry):
            for c in range(EMB // 16):
                sl = pl.ds(16 * c, 16)
                r0_v[r, sl] = r0_v[r, sl] + r1_v[r, sl]
            return carry

        lax.fori_loop(0, n, row, 0)
        pltpu.sync_copy(r0_v, out_hbm.at[pl.ds(base, n)])

    return comb


def _sc_combine(ys, d0, d1):
    return _sc_combine_kernel()(ys, d0, d1)


def _patch_call(p_pad, wp, pb):
    return pl.pallas_call(
        _patch_body,
        grid=(NP_PAD // TILE,),
        in_specs=[
            pl.BlockSpec((TILE, CIN * PATCH * PATCH), lambda i: (i, 0)),
            pl.BlockSpec((EMB, CIN * PATCH * PATCH), lambda i: (0, 0)),
            pl.BlockSpec((1, EMB), lambda i: (0, 0)),
        ],
        out_specs=pl.BlockSpec((TILE, EMB), lambda i: (i, 0)),
        out_shape=jax.ShapeDtypeStruct((NP_PAD, EMB), jnp.float32),
    )(p_pad, wp, pb)


def _router_call(xt_pad, rw_pad, rb_pad):
    return pl.pallas_call(
        _router_body,
        grid=(TP // TILE,),
        in_specs=[
            pl.BlockSpec((TILE, EMB), lambda i: (i, 0)),
            pl.BlockSpec((128, EMB), lambda i: (0, 0)),
            pl.BlockSpec((1, 128), lambda i: (0, 0)),
        ],
        out_specs=pl.BlockSpec((TILE, NEXP), lambda i: (i, 0)),
        out_shape=jax.ShapeDtypeStruct((TP, NEXP), jnp.int32),
    )(xt_pad, rw_pad, rb_pad)


def _ffn_call(eot, xs, ln1_g, ln1_b, Wv, bv, Wo, bo, ln2_g, ln2_b,
              W1, b1, W2, b2):
    mat = lambda i, eot: (eot[i], 0, 0)
    vE = pl.BlockSpec((1, 1, EMB), mat)
    vH = pl.BlockSpec((1, 1, HID), mat)
    r3 = lambda a: a.reshape(NEXP, 1, -1)
    return pl.pallas_call(
        _ffn_body,
        grid_spec=pltpu.PrefetchScalarGridSpec(
            num_scalar_prefetch=1,
            grid=(A_PAD // TILE,),
            in_specs=[
                pl.BlockSpec((TILE, EMB), lambda i, eot: (i, 0)),
                vE,
                vE,
                pl.BlockSpec((1, EMB, EMB), mat),
                vE,
                pl.BlockSpec((1, EMB, EMB), mat),
                vE,
                vE,
                vE,
                pl.BlockSpec((1, HID, EMB), mat),
                vH,
                pl.BlockSpec((1, EMB, HID), mat),
                vE,
            ],
            out_specs=pl.BlockSpec((TILE, EMB), lambda i, eot: (i, 0)),
        ),
        out_shape=jax.ShapeDtypeStruct((A_PAD, EMB), jnp.float32),
        compiler_params=pltpu.CompilerParams(
            dimension_semantics=("arbitrary",)),
    )(eot, xs, r3(ln1_g), r3(ln1_b), Wv.astype(jnp.bfloat16), r3(bv),
      Wo.astype(jnp.bfloat16), r3(bo), r3(ln2_g), r3(ln2_b),
      W1.astype(jnp.bfloat16), r3(b1), W2.astype(jnp.bfloat16), r3(b2))


def _head_call(cls_in, ng, nb, hw_pad, hb_pad):
    return pl.pallas_call(
        _head_body,
        in_specs=[
            pl.BlockSpec((B, EMB), lambda: (0, 0)),
            pl.BlockSpec((1, EMB), lambda: (0, 0)),
            pl.BlockSpec((1, EMB), lambda: (0, 0)),
            pl.BlockSpec((1024, EMB), lambda: (0, 0)),
            pl.BlockSpec((1, 1024), lambda: (0, 0)),
        ],
        out_specs=pl.BlockSpec((B, 1024), lambda: (0, 0)),
        out_shape=jax.ShapeDtypeStruct((B, 1024), jnp.float32),
    )(cls_in, ng, nb, hw_pad, hb_pad)


def kernel(x, patch_W, patch_b, cls_token, pos_embed, router_W, router_b,
           ln1_g, ln1_b, Wv, bv, Wo, bo, ln2_g, ln2_b, W1, b1, W2, b2,
           norm_g, norm_b, head_W, head_b):
    h = IMG // PATCH
    p = x.reshape(B, CIN, h, PATCH, h, PATCH)
    p = jnp.transpose(p, (0, 2, 4, 1, 3, 5)).reshape(B * h * h, CIN * PATCH * PATCH)
    p_pad = jnp.pad(p, ((0, NP_PAD - B * h * h), (0, 0)))
    wp = patch_W.reshape(EMB, CIN * PATCH * PATCH)

    tokens = _patch_call(p_pad, wp, patch_b.reshape(1, EMB))[: B * h * h]
    xt = jnp.concatenate(
        [jnp.broadcast_to(cls_token, (B, 1, EMB)), tokens.reshape(B, h * h, EMB)],
        axis=1) + pos_embed
    xt = jnp.pad(xt.reshape(T, EMB), ((0, TP - T), (0, 0)))

    rw_pad = jnp.pad(router_W, ((0, 128 - NEXP), (0, 0)))
    rb_pad = jnp.pad(router_b, (0, 128 - NEXP)).reshape(1, 128)
    mask = _router_call(xt, rw_pad, rb_pad)
    rows = lax.broadcasted_iota(jnp.int32, (TP, NEXP), 0)
    mask = jnp.where(rows < T, mask, 0)

    counts = jnp.sum(mask, axis=0)
    rank = jnp.cumsum(mask, axis=0) - mask
    padded = ((counts + TILE - 1) // TILE) * TILE
    cum = jnp.cumsum(padded)
    off = cum - padded
    dest = off[None, :] + rank
    valid = mask == 1
    tile_start = jnp.arange(A_PAD // TILE, dtype=jnp.int32) * TILE
    eot = jnp.clip(jnp.searchsorted(cum, tile_start, side="right"),
                   0, NEXP - 1).astype(jnp.int32)

    d0 = jnp.min(jnp.where(valid, dest, A_PAD + 1), axis=1)
    d1 = jnp.max(jnp.where(valid, dest, -1), axis=1)
    trow = jnp.arange(TP, dtype=jnp.int32)
    d0 = jnp.where(trow < T, d0, TRASH).astype(jnp.int32)
    d1 = jnp.where(trow < T, d1, TRASH).astype(jnp.int32)

    xs = _sc_dispatch(xt, d0, d1)
    ys = _ffn_call(eot, xs, ln1_g, ln1_b, Wv, bv, Wo, bo, ln2_g, ln2_b,
                   W1, b1, W2, b2)

    out_tok = _sc_combine(ys, d0, d1)

    cls_rows = jnp.arange(B, dtype=jnp.int32) * NTOK
    cls_in = jnp.take(out_tok, cls_rows, axis=0)
    hw_pad = jnp.pad(head_W, ((0, 1024 - NCLS), (0, 0)))
    hb_pad = jnp.pad(head_b, (0, 1024 - NCLS)).reshape(1, 1024)
    logits = _head_call(cls_in, norm_g.reshape(1, EMB), norm_b.reshape(1, EMB),
                        hw_pad, hb_pad)
    return logits[:, :NCLS]


# speedup vs baseline: 1.0769x; 1.0769x over previous
import functools

import jax
import jax.numpy as jnp
from jax import lax
from jax.experimental import pallas as pl
from jax.experimental.pallas import tpu as pltpu
from jax.experimental.pallas import tpu_sc as plsc

B = 16
IMG = 224
PATCH = 16
CIN = 3
EMB = 384
HID = 1536
NEXP = 8
NCLS = 1000
NPATCH = (IMG // PATCH) ** 2
NTOK = NPATCH + 1
T = B * NTOK
TILE = 128
TP = 3328
NP_PAD = 3200
A_PAD = 58 * TILE
TRASH = A_PAD - 1
NW = 32
NEG = -3.0e38


def _patch_body(p_ref, w_ref, b_ref, o_ref):
    o_ref[...] = lax.dot_general(
        p_ref[...], w_ref[...], (((1,), (1,)), ((), ())),
        preferred_element_type=jnp.float32) + b_ref[0]


def _router_body(x_ref, w_ref, b_ref, mask_ref, rank_ref, cnt_ref, run_ref):
    i = pl.program_id(0)

    @pl.when(i == 0)
    def _init():
        run_ref[...] = jnp.zeros((1, NEXP), jnp.float32)

    logits = lax.dot_general(
        x_ref[...], w_ref[...], (((1,), (1,)), ((), ())),
        preferred_element_type=jnp.float32) + b_ref[0]
    lane = lax.broadcasted_iota(jnp.int32, (TILE, NEXP), 1)
    m0 = jnp.max(logits, axis=1, keepdims=True)
    i0 = jnp.min(jnp.where(logits >= m0, lane, 128), axis=1, keepdims=True)
    oh0 = lane == i0
    l2 = jnp.where(oh0, NEG, logits)
    m1 = jnp.max(l2, axis=1, keepdims=True)
    i1 = jnp.min(jnp.where(l2 >= m1, lane, 128), axis=1, keepdims=True)
    mask = jnp.logical_or(oh0, lane == i1)
    row = lax.broadcasted_iota(jnp.int32, (TILE, NEXP), 0) + i * TILE
    mask = jnp.logical_and(mask, row < T)
    mask_ref[...] = mask.astype(jnp.int32)
    r = lax.broadcasted_iota(jnp.int32, (TILE, TILE), 0)
    c = lax.broadcasted_iota(jnp.int32, (TILE, TILE), 1)
    tri = (r > c).astype(jnp.float32)
    mf = mask.astype(jnp.float32)
    pre = lax.dot_general(tri, mf, (((1,), (0,)), ((), ())),
                          preferred_element_type=jnp.float32)
    rank_ref[...] = (pre + run_ref[...]).astype(jnp.int32)
    run_ref[...] = run_ref[...] + jnp.sum(mf, axis=0, keepdims=True)
    cnt_ref[...] = run_ref[...].astype(jnp.int32)


def _ln(x, g, b, eps=1e-5):
    mu = jnp.mean(x, axis=1, keepdims=True)
    var = jnp.mean((x - mu) ** 2, axis=1, keepdims=True)
    return (x - mu) * lax.rsqrt(var + eps) * g + b


def _ffn_body(eot_ref, xs_ref, g1_ref, c1_ref, wv_ref, bv_ref, wo_ref, bo_ref,
              g2_ref, c2_ref, w1_ref, b1_ref, w2_ref, b2_ref, ys_ref):
    nt = (((1,), (1,)), ((), ()))
    x = xs_ref[...]
    xn = _ln(x, g1_ref[0], c1_ref[0])
    v = lax.dot_general(xn, wv_ref[0], nt,
                        preferred_element_type=jnp.float32) + bv_ref[0]
    attn = lax.dot_general(v, wo_ref[0], nt,
                           preferred_element_type=jnp.float32) + bo_ref[0]
    hmid = x + attn
    hn = _ln(hmid, g2_ref[0], c2_ref[0])
    h1 = lax.dot_general(hn, w1_ref[0], nt,
                         preferred_element_type=jnp.float32) + b1_ref[0]
    h1 = 0.5 * h1 * (1.0 + lax.erf(h1 * 0.7071067811865476))
    m = lax.dot_general(h1, w2_ref[0], nt,
                        preferred_element_type=jnp.float32) + b2_ref[0]
    ys_ref[...] = 0.5 * (hmid + m)


def _head_body(x_ref, g_ref, b_ref, w_ref, hb_ref, o_ref):
    xn = _ln(x_ref[...], g_ref[0], b_ref[0])
    o_ref[...] = lax.dot_general(
        xn, w_ref[...], (((1,), (1,)), ((), ())),
        preferred_element_type=jnp.float32) + hb_ref[0]


@functools.lru_cache(maxsize=None)
def _sc_dispatch_kernel():
    mesh = plsc.VectorSubcoreMesh(core_axis_name="c", subcore_axis_name="s")

    @functools.partial(
        pl.kernel,
        out_type=jax.ShapeDtypeStruct((A_PAD, EMB), jnp.float32),
        mesh=mesh,
        scratch_types=[
            pltpu.VMEM((TP // NW,), jnp.int32),
            pltpu.VMEM((TP // NW,), jnp.int32),
            pltpu.VMEM((TP // NW, EMB), jnp.float32),
            pltpu.SemaphoreType.DMA,
        ],
    )
    def disp(tok_hbm, d0_hbm, d1_hbm, out_hbm, i0_v, i1_v, rows_v, sem):
        n = TP // NW
        wid = lax.axis_index("s") * 2 + lax.axis_index("c")
        base = wid * n
        pltpu.sync_copy(d0_hbm.at[pl.ds(base, n)], i0_v)
        pltpu.sync_copy(d1_hbm.at[pl.ds(base, n)], i1_v)
        pltpu.sync_copy(tok_hbm.at[pl.ds(base, n)], rows_v)
        pltpu.async_copy(rows_v, out_hbm.at[i0_v], sem).wait()
        pltpu.async_copy(rows_v, out_hbm.at[i1_v], sem).wait()

    return disp


def _sc_dispatch(tok, d0, d1):
    return _sc_dispatch_kernel()(tok, d0, d1)


@functools.lru_cache(maxsize=None)
def _sc_combine_kernel():
    mesh = plsc.VectorSubcoreMesh(core_axis_name="c", subcore_axis_name="s")

    @functools.partial(
        pl.kernel,
        out_type=jax.ShapeDtypeStruct((TP, EMB), jnp.float32),
        mesh=mesh,
        scratch_types=[
            pltpu.VMEM((TP // NW,), jnp.int32),
            pltpu.VMEM((TP // NW,), jnp.int32),
            pltpu.VMEM((TP // NW, EMB), jnp.float32),
            pltpu.VMEM((TP // NW, EMB), jnp.float32),
            pltpu.SemaphoreType.DMA,
        ],
    )
    def comb(ys_hbm, d0_hbm, d1_hbm, out_hbm, i0_v, i1_v, r0_v, r1_v, sem):
        n = TP // NW
        wid = lax.axis_index("s") * 2 + lax.axis_index("c")
        base = wid * n
        pltpu.sync_copy(d0_hbm.at[pl.ds(base, n)], i0_v)
        pltpu.sync_copy(d1_hbm.at[pl.ds(base, n)], i1_v)
        pltpu.async_copy(ys_hbm.at[i0_v], r0_v, sem).wait()
        pltpu.async_copy(ys_hbm.at[i1_v], r1_v, sem).wait()

        def row(r, carry):
            for c in range(EMB // 16):
                sl = pl.ds(16 * c, 16)
                r0_v[r, sl] = r0_v[r, sl] + r1_v[r, sl]
            return carry

        lax.fori_loop(0, n, row, 0)
        pltpu.sync_copy(r0_v, out_hbm.at[pl.ds(base, n)])

    return comb


def _sc_combine(ys, d0, d1):
    return _sc_combine_kernel()(ys, d0, d1)


def _patch_call(p_pad, wp, pb):
    return pl.pallas_call(
        _patch_body,
        grid=(NP_PAD // TILE,),
        in_specs=[
            pl.BlockSpec((TILE, CIN * PATCH * PATCH), lambda i: (i, 0)),
            pl.BlockSpec((EMB, CIN * PATCH * PATCH), lambda i: (0, 0)),
            pl.BlockSpec((1, EMB), lambda i: (0, 0)),
        ],
        out_specs=pl.BlockSpec((TILE, EMB), lambda i: (i, 0)),
        out_shape=jax.ShapeDtypeStruct((NP_PAD, EMB), jnp.float32),
    )(p_pad, wp, pb)


def _router_call(xt_pad, rw, rb):
    return pl.pallas_call(
        _router_body,
        grid=(TP // TILE,),
        in_specs=[
            pl.BlockSpec((TILE, EMB), lambda i: (i, 0)),
            pl.BlockSpec((NEXP, EMB), lambda i: (0, 0)),
            pl.BlockSpec((1, NEXP), lambda i: (0, 0)),
        ],
        out_specs=[
            pl.BlockSpec((TILE, NEXP), lambda i: (i, 0)),
            pl.BlockSpec((TILE, NEXP), lambda i: (i, 0)),
            pl.BlockSpec((1, NEXP), lambda i: (0, 0)),
        ],
        out_shape=[
            jax.ShapeDtypeStruct((TP, NEXP), jnp.int32),
            jax.ShapeDtypeStruct((TP, NEXP), jnp.int32),
            jax.ShapeDtypeStruct((1, NEXP), jnp.int32),
        ],
        scratch_shapes=[pltpu.VMEM((1, NEXP), jnp.float32)],
        compiler_params=pltpu.CompilerParams(
            dimension_semantics=("arbitrary",)),
    )(xt_pad, rw, rb)


def _ffn_call(eot, xs, ln1_g, ln1_b, Wv, bv, Wo, bo, ln2_g, ln2_b,
              W1, b1, W2, b2):
    mat = lambda i, eot: (eot[i], 0, 0)
    vE = pl.BlockSpec((1, 1, EMB), mat)
    vH = pl.BlockSpec((1, 1, HID), mat)
    r3 = lambda a: a.reshape(NEXP, 1, -1)
    return pl.pallas_call(
        _ffn_body,
        grid_spec=pltpu.PrefetchScalarGridSpec(
            num_scalar_prefetch=1,
            grid=(A_PAD // TILE,),
            in_specs=[
                pl.BlockSpec((TILE, EMB), lambda i, eot: (i, 0)),
                vE,
                vE,
                pl.BlockSpec((1, EMB, EMB), mat),
                vE,
                pl.BlockSpec((1, EMB, EMB), mat),
                vE,
                vE,
                vE,
                pl.BlockSpec((1, HID, EMB), mat),
                vH,
                pl.BlockSpec((1, EMB, HID), mat),
                vE,
            ],
            out_specs=pl.BlockSpec((TILE, EMB), lambda i, eot: (i, 0)),
        ),
        out_shape=jax.ShapeDtypeStruct((A_PAD, EMB), jnp.float32),
        compiler_params=pltpu.CompilerParams(
            dimension_semantics=("arbitrary",)),
    )(eot, xs, r3(ln1_g), r3(ln1_b), Wv, r3(bv), Wo, r3(bo), r3(ln2_g),
      r3(ln2_b), W1, r3(b1), W2, r3(b2))


def _head_call(cls_in, ng, nb, hw, hb):
    return pl.pallas_call(
        _head_body,
        in_specs=[
            pl.BlockSpec((B, EMB), lambda: (0, 0)),
            pl.BlockSpec((1, EMB), lambda: (0, 0)),
            pl.BlockSpec((1, EMB), lambda: (0, 0)),
            pl.BlockSpec((NCLS, EMB), lambda: (0, 0)),
            pl.BlockSpec((1, NCLS), lambda: (0, 0)),
        ],
        out_specs=pl.BlockSpec((B, NCLS), lambda: (0, 0)),
        out_shape=jax.ShapeDtypeStruct((B, NCLS), jnp.float32),
    )(cls_in, ng, nb, hw, hb)


def kernel(x, patch_W, patch_b, cls_token, pos_embed, router_W, router_b,
           ln1_g, ln1_b, Wv, bv, Wo, bo, ln2_g, ln2_b, W1, b1, W2, b2,
           norm_g, norm_b, head_W, head_b):
    h = IMG // PATCH
    p = x.reshape(B, CIN, h, PATCH, h, PATCH)
    p = jnp.transpose(p, (0, 2, 4, 1, 3, 5)).reshape(B * h * h, CIN * PATCH * PATCH)
    p_pad = jnp.pad(p, ((0, NP_PAD - B * h * h), (0, 0)))
    wp = patch_W.reshape(EMB, CIN * PATCH * PATCH)

    tokens = _patch_call(p_pad, wp, patch_b.reshape(1, EMB))[: B * h * h]
    xt = jnp.concatenate(
        [jnp.broadcast_to(cls_token, (B, 1, EMB)), tokens.reshape(B, h * h, EMB)],
        axis=1) + pos_embed
    xt = jnp.pad(xt.reshape(T, EMB), ((0, TP - T), (0, 0)))

    mask, rank, cnt = _router_call(xt, router_W, router_b.reshape(1, NEXP))

    counts = cnt[0]
    padded = ((counts + TILE - 1) // TILE) * TILE
    cum = jnp.cumsum(padded)
    off = cum - padded
    dest = off[None, :] + rank
    valid = mask == 1
    tile_start = jnp.arange(A_PAD // TILE, dtype=jnp.int32) * TILE
    eot = jnp.clip(jnp.searchsorted(cum, tile_start, side="right"),
                   0, NEXP - 1).astype(jnp.int32)

    d0 = jnp.min(jnp.where(valid, dest, A_PAD + 1), axis=1)
    d1 = jnp.max(jnp.where(valid, dest, -1), axis=1)
    trow = jnp.arange(TP, dtype=jnp.int32)
    d0 = jnp.where(trow < T, d0, TRASH).astype(jnp.int32)
    d1 = jnp.where(trow < T, d1, TRASH).astype(jnp.int32)

    xs = _sc_dispatch(xt, d0, d1)
    ys = _ffn_call(eot, xs, ln1_g, ln1_b, Wv, bv, Wo, bo, ln2_g, ln2_b,
                   W1, b1, W2, b2)

    out_tok = _sc_combine(ys, d0, d1)

    cls_rows = jnp.arange(B, dtype=jnp.int32) * NTOK
    cls_in = jnp.take(out_tok, cls_rows, axis=0)
    return _head_call(cls_in, norm_g.reshape(1, EMB), norm_b.reshape(1, EMB),
                      head_W, head_b.reshape(1, NCLS))


# speedup vs baseline: 1.5853x; 1.4721x over previous
import functools

import jax
import jax.numpy as jnp
from jax import lax
from jax.experimental import pallas as pl
from jax.experimental.pallas import tpu as pltpu
from jax.experimental.pallas import tpu_sc as plsc

B = 16
IMG = 224
PATCH = 16
CIN = 3
EMB = 384
HID = 1536
NEXP = 8
NCLS = 1000
NPATCH = (IMG // PATCH) ** 2
NTOK = NPATCH + 1
T = B * NTOK
TILE = 128
TP = 3328
NP_PAD = 3200
A_PAD = 58 * TILE
TRASH = A_PAD - 1
NW = 32
NEG = -3.0e38


def _patch_body(p_ref, w_ref, b_ref, o_ref):
    o_ref[...] = lax.dot_general(
        p_ref[...], w_ref[...], (((1,), (1,)), ((), ())),
        preferred_element_type=jnp.float32) + b_ref[0]


def _router_body(x_ref, w_ref, b_ref, mask_ref, rank_ref, cnt_ref, run_ref):
    i = pl.program_id(0)

    @pl.when(i == 0)
    def _init():
        run_ref[...] = jnp.zeros((1, NEXP), jnp.float32)

    logits = lax.dot_general(
        x_ref[...], w_ref[...], (((1,), (1,)), ((), ())),
        preferred_element_type=jnp.float32) + b_ref[0]
    lane = lax.broadcasted_iota(jnp.int32, (TILE, NEXP), 1)
    m0 = jnp.max(logits, axis=1, keepdims=True)
    i0 = jnp.min(jnp.where(logits >= m0, lane, 128), axis=1, keepdims=True)
    oh0 = lane == i0
    l2 = jnp.where(oh0, NEG, logits)
    m1 = jnp.max(l2, axis=1, keepdims=True)
    i1 = jnp.min(jnp.where(l2 >= m1, lane, 128), axis=1, keepdims=True)
    mask = jnp.logical_or(oh0, lane == i1)
    row = lax.broadcasted_iota(jnp.int32, (TILE, NEXP), 0) + i * TILE
    mask = jnp.logical_and(mask, row < T)
    mask_ref[...] = mask.astype(jnp.int32)
    r = lax.broadcasted_iota(jnp.int32, (TILE, TILE), 0)
    c = lax.broadcasted_iota(jnp.int32, (TILE, TILE), 1)
    tri = (r > c).astype(jnp.float32)
    mf = mask.astype(jnp.float32)
    pre = lax.dot_general(tri, mf, (((1,), (0,)), ((), ())),
                          preferred_element_type=jnp.float32)
    rank_ref[...] = (pre + run_ref[...]).astype(jnp.int32)
    run_ref[...] = run_ref[...] + jnp.sum(mf, axis=0, keepdims=True)
    cnt_ref[...] = run_ref[...].astype(jnp.int32)


def _ln(x, g, b, eps=1e-5):
    mu = jnp.mean(x, axis=1, keepdims=True)
    var = jnp.mean((x - mu) ** 2, axis=1, keepdims=True)
    return (x - mu) * lax.rsqrt(var + eps) * g + b


def _ffn_body(eot_ref, xs_ref, g1_ref, c1_ref, wv_ref, bv_ref, wo_ref, bo_ref,
              g2_ref, c2_ref, w1_ref, b1_ref, w2_ref, b2_ref, ys_ref):
    nt = (((1,), (1,)), ((), ()))
    x = xs_ref[...]
    xn = _ln(x, g1_ref[0], c1_ref[0])
    v = lax.dot_general(xn, wv_ref[0], nt,
                        preferred_element_type=jnp.float32) + bv_ref[0]
    attn = lax.dot_general(v, wo_ref[0], nt,
                           preferred_element_type=jnp.float32) + bo_ref[0]
    hmid = x + attn
    hn = _ln(hmid, g2_ref[0], c2_ref[0])
    h1 = lax.dot_general(hn, w1_ref[0], nt,
                         preferred_element_type=jnp.float32) + b1_ref[0]
    h1 = 0.5 * h1 * (1.0 + lax.erf(h1 * 0.7071067811865476))
    m = lax.dot_general(h1, w2_ref[0], nt,
                        preferred_element_type=jnp.float32) + b2_ref[0]
    ys_ref[...] = 0.5 * (hmid + m)


def _head_body(x_ref, g_ref, b_ref, w_ref, hb_ref, o_ref):
    xn = _ln(x_ref[...], g_ref[0], b_ref[0])
    o_ref[...] = lax.dot_general(
        xn, w_ref[...], (((1,), (1,)), ((), ())),
        preferred_element_type=jnp.float32) + hb_ref[0]


NGRP = B * (IMG // PATCH)
GPW = NGRP // NW
CU = CIN * PATCH
PF = CIN * PATCH * PATCH


@functools.lru_cache(maxsize=None)
def _sc_im2col_kernel():
    mesh = plsc.VectorSubcoreMesh(core_axis_name="c", subcore_axis_name="s")
    npr = IMG // PATCH

    @functools.partial(
        pl.kernel,
        out_type=jax.ShapeDtypeStruct((NGRP, PATCH, PF), jnp.float32),
        mesh=mesh,
        scratch_types=[
            pltpu.VMEM((2, CU, IMG), jnp.float32),
            pltpu.VMEM((GPW, PATCH, PF), jnp.float32),
            pltpu.SemaphoreType.DMA,
            pltpu.SemaphoreType.DMA,
        ],
    )
    def im2col(x_hbm, p_hbm, slab_v, out_v, sem0, sem1):
        wid = lax.axis_index("s") * 2 + lax.axis_index("c")
        sems = (sem0, sem1)

        def fetch(g, sem):
            gg = wid * GPW + g
            bb = gg // npr
            ii = gg % npr
            for cc in range(CIN):
                pltpu.async_copy(
                    x_hbm.at[pl.ds((bb * CIN + cc) * IMG + PATCH * ii, PATCH), :],
                    slab_v.at[g % 2, pl.ds(cc * PATCH, PATCH), :], sem)

        def drain(g, sem):
            for cc in range(CIN):
                pltpu.make_async_copy(
                    x_hbm.at[pl.ds(0, PATCH), :],
                    slab_v.at[g % 2, pl.ds(cc * PATCH, PATCH), :], sem).wait()

        fetch(0, sems[0])
        for g in range(GPW):
            buf = g % 2
            if g + 1 < GPW:
                fetch(g + 1, sems[1 - buf])
            drain(g, sems[buf])

            def patch_j(j, c2, buf=buf, g=g):
                for cu in range(CU):
                    sl = slab_v[buf, cu, pl.ds(PATCH * j, PATCH)]
                    out_v[g, j, pl.ds(cu * PATCH, PATCH)] = sl
                return c2

            lax.fori_loop(0, npr, patch_j, 0)

        pltpu.sync_copy(out_v, p_hbm.at[pl.ds(wid * GPW, GPW)])

    return im2col


def _sc_im2col(x2d):
    return _sc_im2col_kernel()(x2d)


@functools.lru_cache(maxsize=None)
def _sc_dispatch_kernel():
    mesh = plsc.VectorSubcoreMesh(core_axis_name="c", subcore_axis_name="s")

    @functools.partial(
        pl.kernel,
        out_type=jax.ShapeDtypeStruct((A_PAD, EMB), jnp.float32),
        mesh=mesh,
        scratch_types=[
            pltpu.VMEM((TP // NW,), jnp.int32),
            pltpu.VMEM((TP // NW,), jnp.int32),
            pltpu.VMEM((TP // NW, EMB), jnp.float32),
            pltpu.SemaphoreType.DMA,
        ],
    )
    def disp(tok_hbm, d0_hbm, d1_hbm, out_hbm, i0_v, i1_v, rows_v, sem):
        n = TP // NW
        wid = lax.axis_index("s") * 2 + lax.axis_index("c")
        base = wid * n
        pltpu.sync_copy(d0_hbm.at[pl.ds(base, n)], i0_v)
        pltpu.sync_copy(d1_hbm.at[pl.ds(base, n)], i1_v)
        pltpu.sync_copy(tok_hbm.at[pl.ds(base, n)], rows_v)
        pltpu.async_copy(rows_v, out_hbm.at[i0_v], sem).wait()
        pltpu.async_copy(rows_v, out_hbm.at[i1_v], sem).wait()

    return disp


def _sc_dispatch(tok, d0, d1):
    return _sc_dispatch_kernel()(tok, d0, d1)


@functools.lru_cache(maxsize=None)
def _sc_combine_kernel():
    mesh = plsc.VectorSubcoreMesh(core_axis_name="c", subcore_axis_name="s")

    @functools.partial(
        pl.kernel,
        out_type=jax.ShapeDtypeStruct((TP, EMB), jnp.float32),
        mesh=mesh,
        scratch_types=[
            pltpu.VMEM((TP // NW,), jnp.int32),
            pltpu.VMEM((TP // NW,), jnp.int32),
            pltpu.VMEM((TP // NW, EMB), jnp.float32),
            pltpu.VMEM((TP // NW, EMB), jnp.float32),
            pltpu.SemaphoreType.DMA,
        ],
    )
    def comb(ys_hbm, d0_hbm, d1_hbm, out_hbm, i0_v, i1_v, r0_v, r1_v, sem):
        n = TP // NW
        wid = lax.axis_index("s") * 2 + lax.axis_index("c")
        base = wid * n
        pltpu.sync_copy(d0_hbm.at[pl.ds(base, n)], i0_v)
        pltpu.sync_copy(d1_hbm.at[pl.ds(base, n)], i1_v)
        pltpu.async_copy(ys_hbm.at[i0_v], r0_v, sem).wait()
        pltpu.async_copy(ys_hbm.at[i1_v], r1_v, sem).wait()

        def row(r, carry):
            for c in range(EMB // 16):
                sl = pl.ds(16 * c, 16)
                r0_v[r, sl] = r0_v[r, sl] + r1_v[r, sl]
            return carry

        lax.fori_loop(0, n, row, 0)
        pltpu.sync_copy(r0_v, out_hbm.at[pl.ds(base, n)])

    return comb


def _sc_combine(ys, d0, d1):
    return _sc_combine_kernel()(ys, d0, d1)


PTILE = 112


def _patch_call(p, wp, pb):
    return pl.pallas_call(
        _patch_body,
        grid=(B * NPATCH // PTILE,),
        in_specs=[
            pl.BlockSpec((PTILE, PF), lambda i: (i, 0)),
            pl.BlockSpec((EMB, PF), lambda i: (0, 0)),
            pl.BlockSpec((1, EMB), lambda i: (0, 0)),
        ],
        out_specs=pl.BlockSpec((PTILE, EMB), lambda i: (i, 0)),
        out_shape=jax.ShapeDtypeStruct((B * NPATCH, EMB), jnp.float32),
    )(p, wp, pb)


def _router_call(xt_pad, rw, rb):
    return pl.pallas_call(
        _router_body,
        grid=(TP // TILE,),
        in_specs=[
            pl.BlockSpec((TILE, EMB), lambda i: (i, 0)),
            pl.BlockSpec((NEXP, EMB), lambda i: (0, 0)),
            pl.BlockSpec((1, NEXP), lambda i: (0, 0)),
        ],
        out_specs=[
            pl.BlockSpec((TILE, NEXP), lambda i: (i, 0)),
            pl.BlockSpec((TILE, NEXP), lambda i: (i, 0)),
            pl.BlockSpec((1, NEXP), lambda i: (0, 0)),
        ],
        out_shape=[
            jax.ShapeDtypeStruct((TP, NEXP), jnp.int32),
            jax.ShapeDtypeStruct((TP, NEXP), jnp.int32),
            jax.ShapeDtypeStruct((1, NEXP), jnp.int32),
        ],
        scratch_shapes=[pltpu.VMEM((1, NEXP), jnp.float32)],
        compiler_params=pltpu.CompilerParams(
            dimension_semantics=("arbitrary",)),
    )(xt_pad, rw, rb)


def _ffn_call(eot, xs, ln1_g, ln1_b, Wv, bv, Wo, bo, ln2_g, ln2_b,
              W1, b1, W2, b2):
    mat = lambda i, eot: (eot[i], 0, 0)
    vE = pl.BlockSpec((1, 1, EMB), mat)
    vH = pl.BlockSpec((1, 1, HID), mat)
    r3 = lambda a: a.reshape(NEXP, 1, -1)
    return pl.pallas_call(
        _ffn_body,
        grid_spec=pltpu.PrefetchScalarGridSpec(
            num_scalar_prefetch=1,
            grid=(A_PAD // TILE,),
            in_specs=[
                pl.BlockSpec((TILE, EMB), lambda i, eot: (i, 0)),
                vE,
                vE,
                pl.BlockSpec((1, EMB, EMB), mat),
                vE,
                pl.BlockSpec((1, EMB, EMB), mat),
                vE,
                vE,
                vE,
                pl.BlockSpec((1, HID, EMB), mat),
                vH,
                pl.BlockSpec((1, EMB, HID), mat),
                vE,
            ],
            out_specs=pl.BlockSpec((TILE, EMB), lambda i, eot: (i, 0)),
        ),
        out_shape=jax.ShapeDtypeStruct((A_PAD, EMB), jnp.float32),
        compiler_params=pltpu.CompilerParams(
            dimension_semantics=("arbitrary",)),
    )(eot, xs, r3(ln1_g), r3(ln1_b), Wv, r3(bv), Wo, r3(bo), r3(ln2_g),
      r3(ln2_b), W1, r3(b1), W2, r3(b2))


def _head_call(cls_in, ng, nb, hw, hb):
    return pl.pallas_call(
        _head_body,
        in_specs=[
            pl.BlockSpec((B, EMB), lambda: (0, 0)),
            pl.BlockSpec((1, EMB), lambda: (0, 0)),
            pl.BlockSpec((1, EMB), lambda: (0, 0)),
            pl.BlockSpec((NCLS, EMB), lambda: (0, 0)),
            pl.BlockSpec((1, NCLS), lambda: (0, 0)),
        ],
        out_specs=pl.BlockSpec((B, NCLS), lambda: (0, 0)),
        out_shape=jax.ShapeDtypeStruct((B, NCLS), jnp.float32),
    )(cls_in, ng, nb, hw, hb)


def kernel(x, patch_W, patch_b, cls_token, pos_embed, router_W, router_b,
           ln1_g, ln1_b, Wv, bv, Wo, bo, ln2_g, ln2_b, W1, b1, W2, b2,
           norm_g, norm_b, head_W, head_b):
    h = IMG // PATCH
    p3 = _sc_im2col(x.reshape(B * CIN * IMG, IMG))
    p = p3[:, :IMG // PATCH, :].reshape(B * NPATCH, PF)
    wp = patch_W.reshape(EMB, PF)

    tokens = _patch_call(p, wp, patch_b.reshape(1, EMB))
    xt = jnp.concatenate(
        [jnp.broadcast_to(cls_token, (B, 1, EMB)), tokens.reshape(B, h * h, EMB)],
        axis=1) + pos_embed
    xt = jnp.pad(xt.reshape(T, EMB), ((0, TP - T), (0, 0)))

    mask, rank, cnt = _router_call(xt, router_W, router_b.reshape(1, NEXP))

    counts = cnt[0]
    padded = ((counts + TILE - 1) // TILE) * TILE
    cum = jnp.cumsum(padded)
    off = cum - padded
    dest = off[None, :] + rank
    valid = mask == 1
    tile_start = jnp.arange(A_PAD // TILE, dtype=jnp.int32) * TILE
    eot = jnp.minimum(
        jnp.sum((tile_start[:, None] >= cum[None, :]).astype(jnp.int32),
                axis=1), NEXP - 1).astype(jnp.int32)

    d0 = jnp.min(jnp.where(valid, dest, A_PAD + 1), axis=1)
    d1 = jnp.max(jnp.where(valid, dest, -1), axis=1)
    trow = jnp.arange(TP, dtype=jnp.int32)
    d0 = jnp.where(trow < T, d0, TRASH).astype(jnp.int32)
    d1 = jnp.where(trow < T, d1, TRASH).astype(jnp.int32)

    xs = _sc_dispatch(xt, d0, d1)
    ys = _ffn_call(eot, xs, ln1_g, ln1_b, Wv, bv, Wo, bo, ln2_g, ln2_b,
                   W1, b1, W2, b2)

    out_tok = _sc_combine(ys, d0, d1)

    cls_rows = jnp.arange(B, dtype=jnp.int32) * NTOK
    cls_in = jnp.take(out_tok, cls_rows, axis=0)
    return _head_call(cls_in, norm_g.reshape(1, EMB), norm_b.reshape(1, EMB),
                      head_W, head_b.reshape(1, NCLS))


# speedup vs baseline: 2.0919x; 1.3195x over previous
import functools

import jax
import jax.numpy as jnp
from jax import lax
from jax.experimental import pallas as pl
from jax.experimental.pallas import tpu as pltpu
from jax.experimental.pallas import tpu_sc as plsc

B = 16
IMG = 224
PATCH = 16
CIN = 3
EMB = 384
HID = 1536
NEXP = 8
NCLS = 1000
NPATCH = (IMG // PATCH) ** 2
NTOK = NPATCH + 1
T = B * NTOK
TILE = 256
RT = 256
TP = 3328
A_PAD = 33 * TILE
TRASH = A_PAD - 1
NW = 32
NEG = -3.0e38


def _patch_body(p_ref, w_ref, b_ref, o_ref):
    p2 = p_ref[...].reshape(PG * PATCH, PF)
    o_ref[...] = lax.dot_general(
        p2, w_ref[...], (((1,), (1,)), ((), ())),
        preferred_element_type=jnp.float32) + b_ref[0]


def _router_body(x_ref, w_ref, b_ref, mask_ref, rank_ref, cnt_ref, run_ref):
    i = pl.program_id(0)

    @pl.when(i == 0)
    def _init():
        run_ref[...] = jnp.zeros((1, NEXP), jnp.float32)

    logits = lax.dot_general(
        x_ref[...], w_ref[...], (((1,), (1,)), ((), ())),
        preferred_element_type=jnp.float32) + b_ref[0]
    lane = lax.broadcasted_iota(jnp.int32, (RT, NEXP), 1)
    m0 = jnp.max(logits, axis=1, keepdims=True)
    i0 = jnp.min(jnp.where(logits >= m0, lane, 128), axis=1, keepdims=True)
    oh0 = lane == i0
    l2 = jnp.where(oh0, NEG, logits)
    m1 = jnp.max(l2, axis=1, keepdims=True)
    i1 = jnp.min(jnp.where(l2 >= m1, lane, 128), axis=1, keepdims=True)
    mask = jnp.logical_or(oh0, lane == i1)
    row = lax.broadcasted_iota(jnp.int32, (RT, NEXP), 0) + i * RT
    mask = jnp.logical_and(mask, row < T)
    mask_ref[...] = mask.astype(jnp.int32)
    r = lax.broadcasted_iota(jnp.int32, (RT, RT), 0)
    c = lax.broadcasted_iota(jnp.int32, (RT, RT), 1)
    tri = (r > c).astype(jnp.float32)
    mf = mask.astype(jnp.float32)
    pre = lax.dot_general(tri, mf, (((1,), (0,)), ((), ())),
                          preferred_element_type=jnp.float32)
    rank_ref[...] = (pre + run_ref[...]).astype(jnp.int32)
    run_ref[...] = run_ref[...] + jnp.sum(mf, axis=0, keepdims=True)
    cnt_ref[...] = run_ref[...].astype(jnp.int32)


def _ln(x, g, b, eps=1e-5):
    mu = jnp.mean(x, axis=1, keepdims=True)
    var = jnp.mean((x - mu) ** 2, axis=1, keepdims=True)
    return (x - mu) * lax.rsqrt(var + eps) * g + b


def _ffn_body(eot_ref, xs_ref, g1_ref, c1_ref, wv_ref, bv_ref, wo_ref, bo_ref,
              g2_ref, c2_ref, w1_ref, b1_ref, w2_ref, b2_ref, ys_ref):
    nt = (((1,), (1,)), ((), ()))
    x = xs_ref[...]
    xn = _ln(x, g1_ref[0], c1_ref[0])
    v = lax.dot_general(xn, wv_ref[0], nt,
                        preferred_element_type=jnp.float32) + bv_ref[0]
    attn = lax.dot_general(v, wo_ref[0], nt,
                           preferred_element_type=jnp.float32) + bo_ref[0]
    hmid = x + attn
    hn = _ln(hmid, g2_ref[0], c2_ref[0])
    h1 = lax.dot_general(hn, w1_ref[0], nt,
                         preferred_element_type=jnp.float32) + b1_ref[0]
    h1 = 0.5 * h1 * (1.0 + lax.erf(h1 * 0.7071067811865476))
    m = lax.dot_general(h1, w2_ref[0], nt,
                        preferred_element_type=jnp.float32) + b2_ref[0]
    ys_ref[...] = 0.5 * (hmid + m)


def _head_body(x_ref, g_ref, b_ref, w_ref, hb_ref, o_ref):
    xn = _ln(x_ref[...], g_ref[0], b_ref[0])
    o_ref[...] = lax.dot_general(
        xn, w_ref[...], (((1,), (1,)), ((), ())),
        preferred_element_type=jnp.float32) + hb_ref[0]


NGRP = B * (IMG // PATCH)
GPW = NGRP // NW
CU = CIN * PATCH
PF = CIN * PATCH * PATCH


@functools.lru_cache(maxsize=None)
def _sc_im2col_kernel():
    mesh = plsc.VectorSubcoreMesh(core_axis_name="c", subcore_axis_name="s")
    npr = IMG // PATCH

    @functools.partial(
        pl.kernel,
        out_type=jax.ShapeDtypeStruct((NGRP, PATCH, PF), jnp.float32),
        mesh=mesh,
        scratch_types=[
            pltpu.VMEM((2, CU, IMG), jnp.float32),
            pltpu.VMEM((GPW, PATCH, PF), jnp.float32),
            pltpu.SemaphoreType.DMA,
            pltpu.SemaphoreType.DMA,
        ],
    )
    def im2col(x_hbm, p_hbm, slab_v, out_v, sem0, sem1):
        wid = lax.axis_index("s") * 2 + lax.axis_index("c")
        sems = (sem0, sem1)

        def fetch(g, sem):
            gg = wid * GPW + g
            bb = gg // npr
            ii = gg % npr
            for cc in range(CIN):
                pltpu.async_copy(
                    x_hbm.at[pl.ds((bb * CIN + cc) * IMG + PATCH * ii, PATCH), :],
                    slab_v.at[g % 2, pl.ds(cc * PATCH, PATCH), :], sem)

        def drain(g, sem):
            for cc in range(CIN):
                pltpu.make_async_copy(
                    x_hbm.at[pl.ds(0, PATCH), :],
                    slab_v.at[g % 2, pl.ds(cc * PATCH, PATCH), :], sem).wait()

        fetch(0, sems[0])
        for g in range(GPW):
            buf = g % 2
            if g + 1 < GPW:
                fetch(g + 1, sems[1 - buf])
            drain(g, sems[buf])

            def patch_j(j, c2, buf=buf, g=g):
                for cu in range(CU):
                    sl = slab_v[buf, cu, pl.ds(PATCH * j, PATCH)]
                    out_v[g, j, pl.ds(cu * PATCH, PATCH)] = sl
                return c2

            lax.fori_loop(0, npr, patch_j, 0)

        pltpu.sync_copy(out_v, p_hbm.at[pl.ds(wid * GPW, GPW)])

    return im2col


def _sc_im2col(x2d):
    return _sc_im2col_kernel()(x2d)


@functools.lru_cache(maxsize=None)
def _sc_dispatch_kernel():
    mesh = plsc.VectorSubcoreMesh(core_axis_name="c", subcore_axis_name="s")

    @functools.partial(
        pl.kernel,
        out_type=jax.ShapeDtypeStruct((A_PAD, EMB), jnp.float32),
        mesh=mesh,
        scratch_types=[
            pltpu.VMEM((TP // NW,), jnp.int32),
            pltpu.VMEM((TP // NW,), jnp.int32),
            pltpu.VMEM((TP // NW, EMB), jnp.float32),
            pltpu.SemaphoreType.DMA,
        ],
    )
    def disp(tok_hbm, d0_hbm, d1_hbm, out_hbm, i0_v, i1_v, rows_v, sem):
        n = TP // NW
        wid = lax.axis_index("s") * 2 + lax.axis_index("c")
        base = wid * n
        pltpu.sync_copy(d0_hbm.at[pl.ds(base, n)], i0_v)
        pltpu.sync_copy(d1_hbm.at[pl.ds(base, n)], i1_v)
        pltpu.sync_copy(tok_hbm.at[pl.ds(base, n)], rows_v)
        pltpu.async_copy(rows_v, out_hbm.at[i0_v], sem).wait()
        pltpu.async_copy(rows_v, out_hbm.at[i1_v], sem).wait()

    return disp


def _sc_dispatch(tok, d0, d1):
    return _sc_dispatch_kernel()(tok, d0, d1)


@functools.lru_cache(maxsize=None)
def _sc_combine_kernel():
    mesh = plsc.VectorSubcoreMesh(core_axis_name="c", subcore_axis_name="s")

    @functools.partial(
        pl.kernel,
        out_type=jax.ShapeDtypeStruct((TP, EMB), jnp.float32),
        mesh=mesh,
        scratch_types=[
            pltpu.VMEM((TP // NW,), jnp.int32),
            pltpu.VMEM((TP // NW,), jnp.int32),
            pltpu.VMEM((TP // NW, EMB), jnp.float32),
            pltpu.VMEM((TP // NW, EMB), jnp.float32),
            pltpu.SemaphoreType.DMA,
        ],
    )
    def comb(ys_hbm, d0_hbm, d1_hbm, out_hbm, i0_v, i1_v, r0_v, r1_v, sem):
        n = TP // NW
        wid = lax.axis_index("s") * 2 + lax.axis_index("c")
        base = wid * n
        pltpu.sync_copy(d0_hbm.at[pl.ds(base, n)], i0_v)
        pltpu.sync_copy(d1_hbm.at[pl.ds(base, n)], i1_v)
        pltpu.async_copy(ys_hbm.at[i0_v], r0_v, sem).wait()
        pltpu.async_copy(ys_hbm.at[i1_v], r1_v, sem).wait()

        def row(r, carry):
            for c in range(EMB // 16):
                sl = pl.ds(16 * c, 16)
                r0_v[r, sl] = r0_v[r, sl] + r1_v[r, sl]
            return carry

        lax.fori_loop(0, n, row, 0)
        pltpu.sync_copy(r0_v, out_hbm.at[pl.ds(base, n)])

    return comb


def _sc_combine(ys, d0, d1):
    return _sc_combine_kernel()(ys, d0, d1)


PG = 14


def _patch_call(p3, wp, pb):
    return pl.pallas_call(
        _patch_body,
        grid=(NGRP // PG,),
        in_specs=[
            pl.BlockSpec((PG, PATCH, PF), lambda i: (i, 0, 0)),
            pl.BlockSpec((EMB, PF), lambda i: (0, 0)),
            pl.BlockSpec((1, EMB), lambda i: (0, 0)),
        ],
        out_specs=pl.BlockSpec((PG * PATCH, EMB), lambda i: (i, 0)),
        out_shape=jax.ShapeDtypeStruct((NGRP * PATCH, EMB), jnp.float32),
    )(p3, wp, pb)


def _router_call(xt_pad, rw, rb):
    return pl.pallas_call(
        _router_body,
        grid=(TP // RT,),
        in_specs=[
            pl.BlockSpec((RT, EMB), lambda i: (i, 0)),
            pl.BlockSpec((NEXP, EMB), lambda i: (0, 0)),
            pl.BlockSpec((1, NEXP), lambda i: (0, 0)),
        ],
        out_specs=[
            pl.BlockSpec((RT, NEXP), lambda i: (i, 0)),
            pl.BlockSpec((RT, NEXP), lambda i: (i, 0)),
            pl.BlockSpec((1, NEXP), lambda i: (0, 0)),
        ],
        out_shape=[
            jax.ShapeDtypeStruct((TP, NEXP), jnp.int32),
            jax.ShapeDtypeStruct((TP, NEXP), jnp.int32),
            jax.ShapeDtypeStruct((1, NEXP), jnp.int32),
        ],
        scratch_shapes=[pltpu.VMEM((1, NEXP), jnp.float32)],
        compiler_params=pltpu.CompilerParams(
            dimension_semantics=("arbitrary",)),
    )(xt_pad, rw, rb)


def _ffn_call(eot, xs, ln1_g, ln1_b, Wv, bv, Wo, bo, ln2_g, ln2_b,
              W1, b1, W2, b2):
    mat = lambda i, eot: (eot[i], 0, 0)
    vE = pl.BlockSpec((1, 1, EMB), mat)
    vH = pl.BlockSpec((1, 1, HID), mat)
    r3 = lambda a: a.reshape(NEXP, 1, -1)
    return pl.pallas_call(
        _ffn_body,
        grid_spec=pltpu.PrefetchScalarGridSpec(
            num_scalar_prefetch=1,
            grid=(A_PAD // TILE,),
            in_specs=[
                pl.BlockSpec((TILE, EMB), lambda i, eot: (i, 0)),
                vE,
                vE,
                pl.BlockSpec((1, EMB, EMB), mat),
                vE,
                pl.BlockSpec((1, EMB, EMB), mat),
                vE,
                vE,
                vE,
                pl.BlockSpec((1, HID, EMB), mat),
                vH,
                pl.BlockSpec((1, EMB, HID), mat),
                vE,
            ],
            out_specs=pl.BlockSpec((TILE, EMB), lambda i, eot: (i, 0)),
        ),
        out_shape=jax.ShapeDtypeStruct((A_PAD, EMB), jnp.float32),
        compiler_params=pltpu.CompilerParams(
            dimension_semantics=("arbitrary",)),
    )(eot, xs, r3(ln1_g), r3(ln1_b), Wv, r3(bv), Wo, r3(bo), r3(ln2_g),
      r3(ln2_b), W1, r3(b1), W2, r3(b2))


def _head_call(cls_in, ng, nb, hw, hb):
    return pl.pallas_call(
        _head_body,
        in_specs=[
            pl.BlockSpec((B, EMB), lambda: (0, 0)),
            pl.BlockSpec((1, EMB), lambda: (0, 0)),
            pl.BlockSpec((1, EMB), lambda: (0, 0)),
            pl.BlockSpec((NCLS, EMB), lambda: (0, 0)),
            pl.BlockSpec((1, NCLS), lambda: (0, 0)),
        ],
        out_specs=pl.BlockSpec((B, NCLS), lambda: (0, 0)),
        out_shape=jax.ShapeDtypeStruct((B, NCLS), jnp.float32),
    )(cls_in, ng, nb, hw, hb)


def kernel(x, patch_W, patch_b, cls_token, pos_embed, router_W, router_b,
           ln1_g, ln1_b, Wv, bv, Wo, bo, ln2_g, ln2_b, W1, b1, W2, b2,
           norm_g, norm_b, head_W, head_b):
    h = IMG // PATCH
    p3 = _sc_im2col(x.reshape(B * CIN * IMG, IMG))
    wp = patch_W.reshape(EMB, PF)

    tokens = _patch_call(p3, wp, patch_b.reshape(1, EMB))
    tokens = tokens.reshape(NGRP, PATCH, EMB)[:, :h, :].reshape(B, h * h, EMB)
    xt = jnp.concatenate(
        [jnp.broadcast_to(cls_token, (B, 1, EMB)), tokens],
        axis=1) + pos_embed
    xt = jnp.pad(xt.reshape(T, EMB), ((0, TP - T), (0, 0)))

    mask, rank, cnt = _router_call(xt, router_W, router_b.reshape(1, NEXP))

    counts = cnt[0]
    padded = ((counts + TILE - 1) // TILE) * TILE
    cum = jnp.cumsum(padded)
    off = cum - padded
    dest = off[None, :] + rank
    valid = mask == 1
    tile_start = jnp.arange(A_PAD // TILE, dtype=jnp.int32) * TILE
    eot = jnp.minimum(
        jnp.sum((tile_start[:, None] >= cum[None, :]).astype(jnp.int32),
                axis=1), NEXP - 1).astype(jnp.int32)

    d0 = jnp.min(jnp.where(valid, dest, A_PAD + 1), axis=1)
    d1 = jnp.max(jnp.where(valid, dest, -1), axis=1)
    trow = jnp.arange(TP, dtype=jnp.int32)
    d0 = jnp.where(trow < T, d0, TRASH).astype(jnp.int32)
    d1 = jnp.where(trow < T, d1, TRASH).astype(jnp.int32)

    xs = _sc_dispatch(xt, d0, d1)
    ys = _ffn_call(eot, xs, ln1_g, ln1_b, Wv, bv, Wo, bo, ln2_g, ln2_b,
                   W1, b1, W2, b2)

    out_tok = _sc_combine(ys, d0, d1)

    cls_rows = jnp.arange(B, dtype=jnp.int32) * NTOK
    cls_in = jnp.take(out_tok, cls_rows, axis=0)
    return _head_call(cls_in, norm_g.reshape(1, EMB), norm_b.reshape(1, EMB),
                      head_W, head_b.reshape(1, NCLS))


# speedup vs baseline: 2.1051x; 1.0063x over previous
import functools

import jax
import jax.numpy as jnp
from jax import lax
from jax.experimental import pallas as pl
from jax.experimental.pallas import tpu as pltpu
from jax.experimental.pallas import tpu_sc as plsc

B = 16
IMG = 224
PATCH = 16
CIN = 3
EMB = 384
HID = 1536
NEXP = 8
NCLS = 1000
NPATCH = (IMG // PATCH) ** 2
NTOK = NPATCH + 1
T = B * NTOK
TILE = 256
RT = 256
TP = 3328
A_PAD = 33 * TILE
TRASH = A_PAD - 1
NW = 32
NEG = -3.0e38


def _patch_body(p_ref, w_ref, b_ref, o_ref):
    p2 = p_ref[...].reshape(PG * PATCH, PF)
    o_ref[...] = lax.dot_general(
        p2, w_ref[...], (((1,), (1,)), ((), ())),
        preferred_element_type=jnp.float32) + b_ref[0]


def _router_body(x_ref, w_ref, b_ref, mask_ref, rank_ref, cnt_ref, run_ref):
    i = pl.program_id(0)

    @pl.when(i == 0)
    def _init():
        run_ref[...] = jnp.zeros((1, NEXP), jnp.float32)

    logits = lax.dot_general(
        x_ref[...], w_ref[...], (((1,), (1,)), ((), ())),
        preferred_element_type=jnp.float32) + b_ref[0]
    lane = lax.broadcasted_iota(jnp.int32, (RT, NEXP), 1)
    m0 = jnp.max(logits, axis=1, keepdims=True)
    i0 = jnp.min(jnp.where(logits >= m0, lane, 128), axis=1, keepdims=True)
    oh0 = lane == i0
    l2 = jnp.where(oh0, NEG, logits)
    m1 = jnp.max(l2, axis=1, keepdims=True)
    i1 = jnp.min(jnp.where(l2 >= m1, lane, 128), axis=1, keepdims=True)
    mask = jnp.logical_or(oh0, lane == i1)
    row = lax.broadcasted_iota(jnp.int32, (RT, NEXP), 0) + i * RT
    mask = jnp.logical_and(mask, row < T)
    mask_ref[...] = mask.astype(jnp.int32)
    r = lax.broadcasted_iota(jnp.int32, (RT, RT), 0)
    c = lax.broadcasted_iota(jnp.int32, (RT, RT), 1)
    tri = (r > c).astype(jnp.float32)
    mf = mask.astype(jnp.float32)
    pre = lax.dot_general(tri, mf, (((1,), (0,)), ((), ())),
                          preferred_element_type=jnp.float32)
    rank_ref[...] = (pre + run_ref[...]).astype(jnp.int32)
    run_ref[...] = run_ref[...] + jnp.sum(mf, axis=0, keepdims=True)
    cnt_ref[...] = run_ref[...].astype(jnp.int32)


def _ln(x, g, b, eps=1e-5):
    mu = jnp.mean(x, axis=1, keepdims=True)
    var = jnp.mean((x - mu) ** 2, axis=1, keepdims=True)
    return (x - mu) * lax.rsqrt(var + eps) * g + b


def _ffn_body(eot_ref, xs_ref, g1_ref, c1_ref, wv_ref, bv_ref, wo_ref, bo_ref,
              g2_ref, c2_ref, w1_ref, b1_ref, w2_ref, b2_ref, ys_ref):
    nt = (((1,), (1,)), ((), ()))
    x = xs_ref[...]
    xn = _ln(x, g1_ref[0], c1_ref[0])
    v = lax.dot_general(xn, wv_ref[0], nt,
                        preferred_element_type=jnp.float32) + bv_ref[0]
    attn = lax.dot_general(v, wo_ref[0], nt,
                           preferred_element_type=jnp.float32) + bo_ref[0]
    hmid = x + attn
    hn = _ln(hmid, g2_ref[0], c2_ref[0])
    h1 = lax.dot_general(hn, w1_ref[0], nt,
                         preferred_element_type=jnp.float32) + b1_ref[0]
    h1 = 0.5 * h1 * (1.0 + lax.erf(h1 * 0.7071067811865476))
    m = lax.dot_general(h1, w2_ref[0], nt,
                        preferred_element_type=jnp.float32) + b2_ref[0]
    ys_ref[...] = 0.5 * (hmid + m)


def _head_body(x_ref, g_ref, b_ref, w_ref, hb_ref, o_ref):
    xn = _ln(x_ref[...], g_ref[0], b_ref[0])
    o_ref[...] = lax.dot_general(
        xn, w_ref[...], (((1,), (1,)), ((), ())),
        preferred_element_type=jnp.float32) + hb_ref[0]


NGRP = B * (IMG // PATCH)
GPW = NGRP // NW
CU = CIN * PATCH
PF = CIN * PATCH * PATCH


@functools.lru_cache(maxsize=None)
def _sc_im2col_kernel():
    mesh = plsc.VectorSubcoreMesh(core_axis_name="c", subcore_axis_name="s")
    npr = IMG // PATCH

    @functools.partial(
        pl.kernel,
        out_type=jax.ShapeDtypeStruct((NGRP, PATCH, PF), jnp.float32),
        mesh=mesh,
        scratch_types=[
            pltpu.VMEM((2, CU, IMG), jnp.float32),
            pltpu.VMEM((GPW, PATCH, PF), jnp.float32),
            pltpu.SemaphoreType.DMA,
            pltpu.SemaphoreType.DMA,
        ],
    )
    def im2col(x_hbm, p_hbm, slab_v, out_v, sem0, sem1):
        wid = lax.axis_index("s") * 2 + lax.axis_index("c")
        sems = (sem0, sem1)

        def fetch(g, sem):
            gg = wid * GPW + g
            bb = gg // npr
            ii = gg % npr
            for cc in range(CIN):
                pltpu.async_copy(
                    x_hbm.at[pl.ds((bb * CIN + cc) * IMG + PATCH * ii, PATCH), :],
                    slab_v.at[g % 2, pl.ds(cc * PATCH, PATCH), :], sem)

        def drain(g, sem):
            for cc in range(CIN):
                pltpu.make_async_copy(
                    x_hbm.at[pl.ds(0, PATCH), :],
                    slab_v.at[g % 2, pl.ds(cc * PATCH, PATCH), :], sem).wait()

        fetch(0, sems[0])
        for g in range(GPW):
            buf = g % 2
            if g + 1 < GPW:
                fetch(g + 1, sems[1 - buf])
            drain(g, sems[buf])

            def patch_j(j, c2, buf=buf, g=g):
                for cu in range(CU):
                    sl = slab_v[buf, cu, pl.ds(PATCH * j, PATCH)]
                    out_v[g, j, pl.ds(cu * PATCH, PATCH)] = sl
                return c2

            lax.fori_loop(0, npr, patch_j, 0)

        pltpu.sync_copy(out_v, p_hbm.at[pl.ds(wid * GPW, GPW)])

    return im2col


def _sc_im2col(x2d):
    return _sc_im2col_kernel()(x2d)


@functools.lru_cache(maxsize=None)
def _sc_dispatch_kernel():
    mesh = plsc.VectorSubcoreMesh(core_axis_name="c", subcore_axis_name="s")

    @functools.partial(
        pl.kernel,
        out_type=jax.ShapeDtypeStruct((A_PAD, EMB), jnp.float32),
        mesh=mesh,
        scratch_types=[
            pltpu.VMEM((TP // NW,), jnp.int32),
            pltpu.VMEM((TP // NW,), jnp.int32),
            pltpu.VMEM((TP // NW, EMB), jnp.float32),
            pltpu.SemaphoreType.DMA,
        ],
    )
    def disp(tok_hbm, d0_hbm, d1_hbm, out_hbm, i0_v, i1_v, rows_v, sem):
        n = TP // NW
        wid = lax.axis_index("s") * 2 + lax.axis_index("c")
        base = wid * n
        pltpu.sync_copy(d0_hbm.at[pl.ds(base, n)], i0_v)
        pltpu.sync_copy(d1_hbm.at[pl.ds(base, n)], i1_v)
        pltpu.sync_copy(tok_hbm.at[pl.ds(base, n)], rows_v)
        c0 = pltpu.async_copy(rows_v, out_hbm.at[i0_v], sem)
        c1 = pltpu.async_copy(rows_v, out_hbm.at[i1_v], sem)
        c0.wait()
        c1.wait()

    return disp


def _sc_dispatch(tok, d0, d1):
    return _sc_dispatch_kernel()(tok, d0, d1)


@functools.lru_cache(maxsize=None)
def _sc_combine_kernel():
    mesh = plsc.VectorSubcoreMesh(core_axis_name="c", subcore_axis_name="s")

    @functools.partial(
        pl.kernel,
        out_type=jax.ShapeDtypeStruct((TP, EMB), jnp.float32),
        mesh=mesh,
        scratch_types=[
            pltpu.VMEM((TP // NW,), jnp.int32),
            pltpu.VMEM((TP // NW,), jnp.int32),
            pltpu.VMEM((TP // NW, EMB), jnp.float32),
            pltpu.VMEM((TP // NW, EMB), jnp.float32),
            pltpu.SemaphoreType.DMA,
            pltpu.SemaphoreType.DMA,
        ],
    )
    def comb(ys_hbm, d0_hbm, d1_hbm, out_hbm, i0_v, i1_v, r0_v, r1_v, sem,
             sem2):
        n = TP // NW
        wid = lax.axis_index("s") * 2 + lax.axis_index("c")
        base = wid * n
        pltpu.sync_copy(d0_hbm.at[pl.ds(base, n)], i0_v)
        pltpu.sync_copy(d1_hbm.at[pl.ds(base, n)], i1_v)
        g0 = pltpu.async_copy(ys_hbm.at[i0_v], r0_v, sem)
        g1 = pltpu.async_copy(ys_hbm.at[i1_v], r1_v, sem2)
        g0.wait()
        g1.wait()

        def row(r, carry):
            for c in range(EMB // 16):
                sl = pl.ds(16 * c, 16)
                r0_v[r, sl] = r0_v[r, sl] + r1_v[r, sl]
            return carry

        lax.fori_loop(0, n, row, 0)
        pltpu.sync_copy(r0_v, out_hbm.at[pl.ds(base, n)])

    return comb


def _sc_combine(ys, d0, d1):
    return _sc_combine_kernel()(ys, d0, d1)


PG = 16


def _patch_call(p3, wp, pb):
    return pl.pallas_call(
        _patch_body,
        grid=(NGRP // PG,),
        in_specs=[
            pl.BlockSpec((PG, PATCH, PF), lambda i: (i, 0, 0)),
            pl.BlockSpec((EMB, PF), lambda i: (0, 0)),
            pl.BlockSpec((1, EMB), lambda i: (0, 0)),
        ],
        out_specs=pl.BlockSpec((PG * PATCH, EMB), lambda i: (i, 0)),
        out_shape=jax.ShapeDtypeStruct((NGRP * PATCH, EMB), jnp.float32),
    )(p3, wp, pb)


def _router_call(xt_pad, rw, rb):
    return pl.pallas_call(
        _router_body,
        grid=(TP // RT,),
        in_specs=[
            pl.BlockSpec((RT, EMB), lambda i: (i, 0)),
            pl.BlockSpec((NEXP, EMB), lambda i: (0, 0)),
            pl.BlockSpec((1, NEXP), lambda i: (0, 0)),
        ],
        out_specs=[
            pl.BlockSpec((RT, NEXP), lambda i: (i, 0)),
            pl.BlockSpec((RT, NEXP), lambda i: (i, 0)),
            pl.BlockSpec((1, NEXP), lambda i: (0, 0)),
        ],
        out_shape=[
            jax.ShapeDtypeStruct((TP, NEXP), jnp.int32),
            jax.ShapeDtypeStruct((TP, NEXP), jnp.int32),
            jax.ShapeDtypeStruct((1, NEXP), jnp.int32),
        ],
        scratch_shapes=[pltpu.VMEM((1, NEXP), jnp.float32)],
        compiler_params=pltpu.CompilerParams(
            dimension_semantics=("arbitrary",)),
    )(xt_pad, rw, rb)


def _ffn_call(eot, xs, ln1_g, ln1_b, Wv, bv, Wo, bo, ln2_g, ln2_b,
              W1, b1, W2, b2):
    mat = lambda i, eot: (eot[i], 0, 0)
    vE = pl.BlockSpec((1, 1, EMB), mat)
    vH = pl.BlockSpec((1, 1, HID), mat)
    r3 = lambda a: a.reshape(NEXP, 1, -1)
    return pl.pallas_call(
        _ffn_body,
        grid_spec=pltpu.PrefetchScalarGridSpec(
            num_scalar_prefetch=1,
            grid=(A_PAD // TILE,),
            in_specs=[
                pl.BlockSpec((TILE, EMB), lambda i, eot: (i, 0)),
                vE,
                vE,
                pl.BlockSpec((1, EMB, EMB), mat),
                vE,
                pl.BlockSpec((1, EMB, EMB), mat),
                vE,
                vE,
                vE,
                pl.BlockSpec((1, HID, EMB), mat),
                vH,
                pl.BlockSpec((1, EMB, HID), mat),
                vE,
            ],
            out_specs=pl.BlockSpec((TILE, EMB), lambda i, eot: (i, 0)),
        ),
        out_shape=jax.ShapeDtypeStruct((A_PAD, EMB), jnp.float32),
        compiler_params=pltpu.CompilerParams(
            dimension_semantics=("arbitrary",)),
    )(eot, xs, r3(ln1_g), r3(ln1_b), Wv, r3(bv), Wo, r3(bo), r3(ln2_g),
      r3(ln2_b), W1, r3(b1), W2, r3(b2))


def _head_call(cls_in, ng, nb, hw, hb):
    return pl.pallas_call(
        _head_body,
        in_specs=[
            pl.BlockSpec((B, EMB), lambda: (0, 0)),
            pl.BlockSpec((1, EMB), lambda: (0, 0)),
            pl.BlockSpec((1, EMB), lambda: (0, 0)),
            pl.BlockSpec((NCLS, EMB), lambda: (0, 0)),
            pl.BlockSpec((1, NCLS), lambda: (0, 0)),
        ],
        out_specs=pl.BlockSpec((B, NCLS), lambda: (0, 0)),
        out_shape=jax.ShapeDtypeStruct((B, NCLS), jnp.float32),
    )(cls_in, ng, nb, hw, hb)


def kernel(x, patch_W, patch_b, cls_token, pos_embed, router_W, router_b,
           ln1_g, ln1_b, Wv, bv, Wo, bo, ln2_g, ln2_b, W1, b1, W2, b2,
           norm_g, norm_b, head_W, head_b):
    h = IMG // PATCH
    p3 = _sc_im2col(x.reshape(B * CIN * IMG, IMG))
    wp = patch_W.reshape(EMB, PF)

    tokens = _patch_call(p3, wp, patch_b.reshape(1, EMB))
    tokens = tokens.reshape(NGRP, PATCH, EMB)[:, :h, :].reshape(B, h * h, EMB)
    xt = jnp.concatenate(
        [jnp.broadcast_to(cls_token, (B, 1, EMB)), tokens],
        axis=1) + pos_embed
    xt = jnp.pad(xt.reshape(T, EMB), ((0, TP - T), (0, 0)))

    mask, rank, cnt = _router_call(xt, router_W, router_b.reshape(1, NEXP))

    counts = cnt[0]
    padded = ((counts + TILE - 1) // TILE) * TILE
    cum = jnp.cumsum(padded)
    off = cum - padded
    dest = off[None, :] + rank
    valid = mask == 1
    tile_start = jnp.arange(A_PAD // TILE, dtype=jnp.int32) * TILE
    eot = jnp.minimum(
        jnp.sum((tile_start[:, None] >= cum[None, :]).astype(jnp.int32),
                axis=1), NEXP - 1).astype(jnp.int32)

    d0 = jnp.min(jnp.where(valid, dest, A_PAD + 1), axis=1)
    d1 = jnp.max(jnp.where(valid, dest, -1), axis=1)
    trow = jnp.arange(TP, dtype=jnp.int32)
    d0 = jnp.where(trow < T, d0, TRASH).astype(jnp.int32)
    d1 = jnp.where(trow < T, d1, TRASH).astype(jnp.int32)

    xs = _sc_dispatch(xt, d0, d1)
    ys = _ffn_call(eot, xs, ln1_g, ln1_b, Wv, bv, Wo, bo, ln2_g, ln2_b,
                   W1, b1, W2, b2)

    out_tok = _sc_combine(ys, d0, d1)

    cls_rows = jnp.arange(B, dtype=jnp.int32) * NTOK
    cls_in = jnp.take(out_tok, cls_rows, axis=0)
    return _head_call(cls_in, norm_g.reshape(1, EMB), norm_b.reshape(1, EMB),
                      head_W, head_b.reshape(1, NCLS))


# speedup vs baseline: 2.1622x; 1.0271x over previous
import functools

import jax
import jax.numpy as jnp
from jax import lax
from jax.experimental import pallas as pl
from jax.experimental.pallas import tpu as pltpu
from jax.experimental.pallas import tpu_sc as plsc

B = 16
IMG = 224
PATCH = 16
CIN = 3
EMB = 384
HID = 1536
NEXP = 8
NCLS = 1000
NPATCH = (IMG // PATCH) ** 2
NTOK = NPATCH + 1
T = B * NTOK
TILE = 256
RT = 256
TP = 3328
A_PAD = 33 * TILE
TRASH = A_PAD - 1
NW = 32
NEG = -3.0e38


def _patch_body(p_ref, w_ref, b_ref, o_ref):
    p2 = p_ref[...].reshape(PG * PATCH, PF)
    o_ref[...] = lax.dot_general(
        p2, w_ref[...], (((1,), (1,)), ((), ())),
        preferred_element_type=jnp.float32) + b_ref[0]


def _router_body(x_ref, w_ref, b_ref, mask_ref, rank_ref, cnt_ref, run_ref):
    i = pl.program_id(0)

    @pl.when(i == 0)
    def _init():
        run_ref[...] = jnp.zeros((1, NEXP), jnp.float32)

    logits = lax.dot_general(
        x_ref[...], w_ref[...], (((1,), (1,)), ((), ())),
        preferred_element_type=jnp.float32) + b_ref[0]
    lane = lax.broadcasted_iota(jnp.int32, (RT, NEXP), 1)
    m0 = jnp.max(logits, axis=1, keepdims=True)
    i0 = jnp.min(jnp.where(logits >= m0, lane, 128), axis=1, keepdims=True)
    oh0 = lane == i0
    l2 = jnp.where(oh0, NEG, logits)
    m1 = jnp.max(l2, axis=1, keepdims=True)
    i1 = jnp.min(jnp.where(l2 >= m1, lane, 128), axis=1, keepdims=True)
    mask = jnp.logical_or(oh0, lane == i1)
    row = lax.broadcasted_iota(jnp.int32, (RT, NEXP), 0) + i * RT
    mask = jnp.logical_and(mask, row < T)
    mask_ref[...] = mask.astype(jnp.int32)
    r = lax.broadcasted_iota(jnp.int32, (RT, RT), 0)
    c = lax.broadcasted_iota(jnp.int32, (RT, RT), 1)
    tri = (r > c).astype(jnp.float32)
    mf = mask.astype(jnp.float32)
    pre = lax.dot_general(tri, mf, (((1,), (0,)), ((), ())),
                          preferred_element_type=jnp.float32)
    rank_ref[...] = (pre + run_ref[...]).astype(jnp.int32)
    run_ref[...] = run_ref[...] + jnp.sum(mf, axis=0, keepdims=True)
    cnt_ref[...] = run_ref[...].astype(jnp.int32)


def _ln(x, g, b, eps=1e-5):
    mu = jnp.mean(x, axis=1, keepdims=True)
    var = jnp.mean((x - mu) ** 2, axis=1, keepdims=True)
    return (x - mu) * lax.rsqrt(var + eps) * g + b


def _ffn_body(eot_ref, xs_ref, g1_ref, c1_ref, wv_ref, bv_ref, wo_ref, bo_ref,
              g2_ref, c2_ref, w1_ref, b1_ref, w2_ref, b2_ref, ys_ref):
    nt = (((1,), (1,)), ((), ()))
    x = xs_ref[...]
    xn = _ln(x, g1_ref[0], c1_ref[0])
    v = lax.dot_general(xn, wv_ref[0], nt,
                        preferred_element_type=jnp.float32) + bv_ref[0]
    attn = lax.dot_general(v, wo_ref[0], nt,
                           preferred_element_type=jnp.float32) + bo_ref[0]
    hmid = x + attn
    hn = _ln(hmid, g2_ref[0], c2_ref[0])
    h1 = lax.dot_general(hn, w1_ref[0], nt,
                         preferred_element_type=jnp.float32) + b1_ref[0]
    h1 = 0.5 * h1 * (1.0 + lax.erf(h1 * 0.7071067811865476))
    m = lax.dot_general(h1, w2_ref[0], nt,
                        preferred_element_type=jnp.float32) + b2_ref[0]
    ys_ref[...] = 0.5 * (hmid + m)


def _head_body(x_ref, g_ref, b_ref, w_ref, hb_ref, o_ref):
    xn = _ln(x_ref[...], g_ref[0], b_ref[0])
    o_ref[...] = lax.dot_general(
        xn, w_ref[...], (((1,), (1,)), ((), ())),
        preferred_element_type=jnp.float32) + hb_ref[0]


NGRP = B * (IMG // PATCH)
GPW = NGRP // NW
CU = CIN * PATCH
PF = CIN * PATCH * PATCH


@functools.lru_cache(maxsize=None)
def _sc_im2col_kernel():
    mesh = plsc.VectorSubcoreMesh(core_axis_name="c", subcore_axis_name="s")
    npr = IMG // PATCH

    @functools.partial(
        pl.kernel,
        out_type=jax.ShapeDtypeStruct((NGRP, PATCH, PF), jnp.float32),
        mesh=mesh,
        scratch_types=[
            pltpu.VMEM((2, CU, IMG), jnp.float32),
            pltpu.VMEM((GPW, PATCH, PF), jnp.float32),
            pltpu.SemaphoreType.DMA,
            pltpu.SemaphoreType.DMA,
        ],
    )
    def im2col(x_hbm, p_hbm, slab_v, out_v, sem0, sem1):
        wid = lax.axis_index("s") * 2 + lax.axis_index("c")
        sems = (sem0, sem1)

        def fetch(g, sem):
            gg = wid * GPW + g
            bb = gg // npr
            ii = gg % npr
            for cc in range(CIN):
                pltpu.async_copy(
                    x_hbm.at[pl.ds((bb * CIN + cc) * IMG + PATCH * ii, PATCH), :],
                    slab_v.at[g % 2, pl.ds(cc * PATCH, PATCH), :], sem)

        def drain(g, sem):
            for cc in range(CIN):
                pltpu.make_async_copy(
                    x_hbm.at[pl.ds(0, PATCH), :],
                    slab_v.at[g % 2, pl.ds(cc * PATCH, PATCH), :], sem).wait()

        fetch(0, sems[0])
        for g in range(GPW):
            buf = g % 2
            if g + 1 < GPW:
                fetch(g + 1, sems[1 - buf])
            drain(g, sems[buf])

            @plsc.parallel_loop(0, npr, unroll=2)
            def patch_j(j, buf=buf, g=g):
                for cu in range(CU):
                    sl = slab_v[buf, cu, pl.ds(PATCH * j, PATCH)]
                    out_v[g, j, pl.ds(cu * PATCH, PATCH)] = sl

        pltpu.sync_copy(out_v, p_hbm.at[pl.ds(wid * GPW, GPW)])

    return im2col


def _sc_im2col(x2d):
    return _sc_im2col_kernel()(x2d)


@functools.lru_cache(maxsize=None)
def _sc_dispatch_kernel():
    mesh = plsc.VectorSubcoreMesh(core_axis_name="c", subcore_axis_name="s")

    @functools.partial(
        pl.kernel,
        out_type=jax.ShapeDtypeStruct((A_PAD, EMB), jnp.float32),
        mesh=mesh,
        scratch_types=[
            pltpu.VMEM((TP // NW,), jnp.int32),
            pltpu.VMEM((TP // NW,), jnp.int32),
            pltpu.VMEM((TP // NW, EMB), jnp.float32),
            pltpu.SemaphoreType.DMA,
        ],
    )
    def disp(tok_hbm, d0_hbm, d1_hbm, out_hbm, i0_v, i1_v, rows_v, sem):
        n = TP // NW
        wid = lax.axis_index("s") * 2 + lax.axis_index("c")
        base = wid * n
        pltpu.sync_copy(d0_hbm.at[pl.ds(base, n)], i0_v)
        pltpu.sync_copy(d1_hbm.at[pl.ds(base, n)], i1_v)
        pltpu.sync_copy(tok_hbm.at[pl.ds(base, n)], rows_v)
        c0 = pltpu.async_copy(rows_v, out_hbm.at[i0_v], sem)
        c1 = pltpu.async_copy(rows_v, out_hbm.at[i1_v], sem)
        c0.wait()
        c1.wait()

    return disp


def _sc_dispatch(tok, d0, d1):
    return _sc_dispatch_kernel()(tok, d0, d1)


@functools.lru_cache(maxsize=None)
def _sc_combine_kernel():
    mesh = plsc.VectorSubcoreMesh(core_axis_name="c", subcore_axis_name="s")

    @functools.partial(
        pl.kernel,
        out_type=jax.ShapeDtypeStruct((TP, EMB), jnp.float32),
        mesh=mesh,
        scratch_types=[
            pltpu.VMEM((TP // NW,), jnp.int32),
            pltpu.VMEM((TP // NW,), jnp.int32),
            pltpu.VMEM((TP // NW, EMB), jnp.float32),
            pltpu.VMEM((TP // NW, EMB), jnp.float32),
            pltpu.SemaphoreType.DMA,
            pltpu.SemaphoreType.DMA,
        ],
    )
    def comb(ys_hbm, d0_hbm, d1_hbm, out_hbm, i0_v, i1_v, r0_v, r1_v, sem,
             sem2):
        n = TP // NW
        wid = lax.axis_index("s") * 2 + lax.axis_index("c")
        base = wid * n
        pltpu.sync_copy(d0_hbm.at[pl.ds(base, n)], i0_v)
        pltpu.sync_copy(d1_hbm.at[pl.ds(base, n)], i1_v)
        g0 = pltpu.async_copy(ys_hbm.at[i0_v], r0_v, sem)
        g1 = pltpu.async_copy(ys_hbm.at[i1_v], r1_v, sem2)
        g0.wait()
        g1.wait()

        @plsc.parallel_loop(0, n, unroll=2)
        def row(r):
            for c in range(EMB // 16):
                sl = pl.ds(16 * c, 16)
                r0_v[r, sl] = r0_v[r, sl] + r1_v[r, sl]
        pltpu.sync_copy(r0_v, out_hbm.at[pl.ds(base, n)])

    return comb


def _sc_combine(ys, d0, d1):
    return _sc_combine_kernel()(ys, d0, d1)


PG = 16


def _patch_call(p3, wp, pb):
    return pl.pallas_call(
        _patch_body,
        grid=(NGRP // PG,),
        in_specs=[
            pl.BlockSpec((PG, PATCH, PF), lambda i: (i, 0, 0)),
            pl.BlockSpec((EMB, PF), lambda i: (0, 0)),
            pl.BlockSpec((1, EMB), lambda i: (0, 0)),
        ],
        out_specs=pl.BlockSpec((PG * PATCH, EMB), lambda i: (i, 0)),
        out_shape=jax.ShapeDtypeStruct((NGRP * PATCH, EMB), jnp.float32),
    )(p3, wp, pb)


def _router_call(xt_pad, rw, rb):
    return pl.pallas_call(
        _router_body,
        grid=(TP // RT,),
        in_specs=[
            pl.BlockSpec((RT, EMB), lambda i: (i, 0)),
            pl.BlockSpec((NEXP, EMB), lambda i: (0, 0)),
            pl.BlockSpec((1, NEXP), lambda i: (0, 0)),
        ],
        out_specs=[
            pl.BlockSpec((RT, NEXP), lambda i: (i, 0)),
            pl.BlockSpec((RT, NEXP), lambda i: (i, 0)),
            pl.BlockSpec((1, NEXP), lambda i: (0, 0)),
        ],
        out_shape=[
            jax.ShapeDtypeStruct((TP, NEXP), jnp.int32),
            jax.ShapeDtypeStruct((TP, NEXP), jnp.int32),
            jax.ShapeDtypeStruct((1, NEXP), jnp.int32),
        ],
        scratch_shapes=[pltpu.VMEM((1, NEXP), jnp.float32)],
        compiler_params=pltpu.CompilerParams(
            dimension_semantics=("arbitrary",)),
    )(xt_pad, rw, rb)


def _ffn_call(eot, xs, ln1_g, ln1_b, Wv, bv, Wo, bo, ln2_g, ln2_b,
              W1, b1, W2, b2):
    mat = lambda i, eot: (eot[i], 0, 0)
    vE = pl.BlockSpec((1, 1, EMB), mat)
    vH = pl.BlockSpec((1, 1, HID), mat)
    r3 = lambda a: a.reshape(NEXP, 1, -1)
    return pl.pallas_call(
        _ffn_body,
        grid_spec=pltpu.PrefetchScalarGridSpec(
            num_scalar_prefetch=1,
            grid=(A_PAD // TILE,),
            in_specs=[
                pl.BlockSpec((TILE, EMB), lambda i, eot: (i, 0)),
                vE,
                vE,
                pl.BlockSpec((1, EMB, EMB), mat),
                vE,
                pl.BlockSpec((1, EMB, EMB), mat),
                vE,
                vE,
                vE,
                pl.BlockSpec((1, HID, EMB), mat),
                vH,
                pl.BlockSpec((1, EMB, HID), mat),
                vE,
            ],
            out_specs=pl.BlockSpec((TILE, EMB), lambda i, eot: (i, 0)),
        ),
        out_shape=jax.ShapeDtypeStruct((A_PAD, EMB), jnp.float32),
        compiler_params=pltpu.CompilerParams(
            dimension_semantics=("arbitrary",)),
    )(eot, xs, r3(ln1_g), r3(ln1_b), Wv, r3(bv), Wo, r3(bo), r3(ln2_g),
      r3(ln2_b), W1, r3(b1), W2, r3(b2))


def _head_call(cls_in, ng, nb, hw, hb):
    return pl.pallas_call(
        _head_body,
        in_specs=[
            pl.BlockSpec((B, EMB), lambda: (0, 0)),
            pl.BlockSpec((1, EMB), lambda: (0, 0)),
            pl.BlockSpec((1, EMB), lambda: (0, 0)),
            pl.BlockSpec((NCLS, EMB), lambda: (0, 0)),
            pl.BlockSpec((1, NCLS), lambda: (0, 0)),
        ],
        out_specs=pl.BlockSpec((B, NCLS), lambda: (0, 0)),
        out_shape=jax.ShapeDtypeStruct((B, NCLS), jnp.float32),
    )(cls_in, ng, nb, hw, hb)


def kernel(x, patch_W, patch_b, cls_token, pos_embed, router_W, router_b,
           ln1_g, ln1_b, Wv, bv, Wo, bo, ln2_g, ln2_b, W1, b1, W2, b2,
           norm_g, norm_b, head_W, head_b):
    h = IMG // PATCH
    p3 = _sc_im2col(x.reshape(B * CIN * IMG, IMG))
    wp = patch_W.reshape(EMB, PF)

    tokens = _patch_call(p3, wp, patch_b.reshape(1, EMB))
    tokens = tokens.reshape(NGRP, PATCH, EMB)[:, :h, :].reshape(B, h * h, EMB)
    xt = jnp.concatenate(
        [jnp.broadcast_to(cls_token, (B, 1, EMB)), tokens],
        axis=1) + pos_embed
    xt = jnp.pad(xt.reshape(T, EMB), ((0, TP - T), (0, 0)))

    mask, rank, cnt = _router_call(xt, router_W, router_b.reshape(1, NEXP))

    counts = cnt[0]
    padded = ((counts + TILE - 1) // TILE) * TILE
    cum = jnp.cumsum(padded)
    off = cum - padded
    dest = off[None, :] + rank
    valid = mask == 1
    tile_start = jnp.arange(A_PAD // TILE, dtype=jnp.int32) * TILE
    eot = jnp.minimum(
        jnp.sum((tile_start[:, None] >= cum[None, :]).astype(jnp.int32),
                axis=1), NEXP - 1).astype(jnp.int32)

    d0 = jnp.min(jnp.where(valid, dest, A_PAD + 1), axis=1)
    d1 = jnp.max(jnp.where(valid, dest, -1), axis=1)
    trow = jnp.arange(TP, dtype=jnp.int32)
    d0 = jnp.where(trow < T, d0, TRASH).astype(jnp.int32)
    d1 = jnp.where(trow < T, d1, TRASH).astype(jnp.int32)

    xs = _sc_dispatch(xt, d0, d1)
    ys = _ffn_call(eot, xs, ln1_g, ln1_b, Wv, bv, Wo, bo, ln2_g, ln2_b,
                   W1, b1, W2, b2)

    out_tok = _sc_combine(ys, d0, d1)

    cls_rows = jnp.arange(B, dtype=jnp.int32) * NTOK
    cls_in = jnp.take(out_tok, cls_rows, axis=0)
    return _head_call(cls_in, norm_g.reshape(1, EMB), norm_b.reshape(1, EMB),
                      head_W, head_b.reshape(1, NCLS))
